# baseline (speedup 1.0000x reference)
import jax
import jax.numpy as jnp
from jax.experimental import pallas as pl
from jax.experimental.pallas import tpu as pltpu

_CRACK = 1
_KPAD = 128


def _issue(lg_ref, row_ref, colc_ref, bb, buf, sem):
    for k in range(_KPAD):
        r = row_ref[bb, k]
        cs = pl.multiple_of(colc_ref[bb, k], 128)
        pltpu.make_async_copy(
            lg_ref.at[bb, :, r, pl.ds(cs, 128)],
            buf.at[4 * k:4 * k + 4, 0, :],
            sem,
        ).start()


def _wait_all(lg_ref, buf, sem):
    for k in range(_KPAD):
        pltpu.make_async_copy(
            lg_ref.at[0, :, 0, pl.ds(0, 128)],
            buf.at[4 * k:4 * k + 4, 0, :],
            sem,
        ).wait()


def _epilogue(buf, col_ref, tgt_ref, vw_ref, out_ref):
    t = buf[...].reshape(_KPAD, 4, 128)
    m = jnp.max(t, axis=1, keepdims=True)
    e = jnp.exp(t - m)
    s = jnp.sum(e, axis=1, keepdims=True)
    crack = e[:, _CRACK:_CRACK + 1, :] * (1.0 / s)
    lane = jax.lax.broadcasted_iota(jnp.int32, crack.shape, 2)
    sel = lane == col_ref[...]
    d = crack - tgt_ref[...]
    term = jnp.where(sel, vw_ref[...] * d * d, 0.0)
    out_ref[...] = jnp.sum(term, axis=(0, 2), keepdims=True)[0]


def _loss_kernel(row_ref, colc_ref, lg_ref, col_ref, tgt_ref, vw_ref,
                 out_ref, gb0, gb1, sem0, sem1):
    nb = row_ref.shape[0]
    b = pl.program_id(0)

    @pl.when(b == 0)
    def _():
        _issue(lg_ref, row_ref, colc_ref, 0, gb0, sem0)

    def _step(cur_buf, cur_sem, nxt_buf, nxt_sem):
        @pl.when(b + 1 < nb)
        def _():
            _issue(lg_ref, row_ref, colc_ref, b + 1, nxt_buf, nxt_sem)
        _wait_all(lg_ref, cur_buf, cur_sem)
        _epilogue(cur_buf, col_ref, tgt_ref, vw_ref, out_ref)

    @pl.when((b & 1) == 0)
    def _():
        _step(gb0, sem0, gb1, sem1)

    @pl.when((b & 1) == 1)
    def _():
        _step(gb1, sem1, gb0, sem0)


def kernel(logits, masks, term_idx, term_tgt, term_valid, term_count):
    del masks
    b_n, c_n, h_n, w_n = logits.shape
    k_n = term_idx.shape[1]
    pad = _KPAD - k_n
    idx = jnp.pad(term_idx, ((0, 0), (0, pad)))
    tgt = jnp.pad(term_tgt, ((0, 0), (0, pad)))
    valid = jnp.pad(term_valid, ((0, 0), (0, pad)))
    rows = (idx // w_n).astype(jnp.int32)
    col = (idx % w_n).astype(jnp.int32)
    colc = col & ~jnp.int32(127)
    lanec = (col & 127).reshape(b_n, _KPAD, 1, 1)
    vw = (valid / (term_count * b_n)[:, None]).reshape(b_n, _KPAD, 1, 1)
    tgt = tgt.reshape(b_n, _KPAD, 1, 1)
    out = pl.pallas_call(
        _loss_kernel,
        grid=(b_n,),
        in_specs=[
            pl.BlockSpec(memory_space=pltpu.SMEM),
            pl.BlockSpec(memory_space=pltpu.SMEM),
            pl.BlockSpec(memory_space=pl.ANY),
            pl.BlockSpec((None, _KPAD, 1, 1), lambda b: (b, 0, 0, 0)),
            pl.BlockSpec((None, _KPAD, 1, 1), lambda b: (b, 0, 0, 0)),
            pl.BlockSpec((None, _KPAD, 1, 1), lambda b: (b, 0, 0, 0)),
        ],
        out_specs=pl.BlockSpec((None, 1, 1), lambda b: (b, 0, 0)),
        out_shape=jax.ShapeDtypeStruct((b_n, 1, 1), jnp.float32),
        scratch_shapes=[
            pltpu.VMEM((4 * _KPAD, 1, 128), jnp.float32),
            pltpu.VMEM((4 * _KPAD, 1, 128), jnp.float32),
            pltpu.SemaphoreType.DMA,
            pltpu.SemaphoreType.DMA,
        ],
        compiler_params=pltpu.CompilerParams(
            dimension_semantics=("arbitrary",),
        ),
        name="topology_loss",
    )(rows, colc, logits, lanec, tgt, vw)
    return jnp.sum(out)
